# baseline (speedup 1.0000x reference)
import functools

import jax
import jax.numpy as jnp
from jax import lax
from jax.experimental import pallas as pl
from jax.experimental.pallas import tpu as pltpu
from jax.experimental.pallas import tpu_sc as plsc

_LANES = 128
_CL = 64
_BLK = 256
_KG = 4


def _make_sc_deg(n_pad, n_chunks):
    mesh = plsc.VectorSubcoreMesh(core_axis_name="c", subcore_axis_name="s")
    rows_per_tile = n_pad // 16
    assert n_chunks % _KG == 0

    @functools.partial(
        pl.kernel,
        out_type=jax.ShapeDtypeStruct((2, n_pad, _LANES), jnp.float32),
        mesh=mesh,
        scratch_types=(
            [pltpu.VMEM_SHARED((n_pad, _LANES), jnp.float32)]
            + [pltpu.VMEM((_LANES,), jnp.int32) for _ in range(4)]
            + [pltpu.VMEM((_LANES, _LANES), jnp.float32)]
            + [pltpu.SemaphoreType.DMA for _ in range(4)]
        ),
    )
    def deg(dst_hbm, zeros_hbm, ones_hbm, out_hbm, acc,
            db0, db1, db2, db3, ones_v, is0, is1, is2, is3):
        c = lax.axis_index("c")
        s = lax.axis_index("s")
        row0 = s * rows_per_tile
        my_dst = dst_hbm.at[c, s]
        dstbs = (db0, db1, db2, db3)
        isems = (is0, is1, is2, is3)

        def issue_idx(j, q):
            pltpu.async_copy(my_dst.at[j], dstbs[q], isems[q])

        def wait_idx(q):
            pltpu.make_async_copy(my_dst.at[0], dstbs[q], isems[q]).wait()

        for q in range(4):
            issue_idx(q, q)
        pltpu.sync_copy(zeros_hbm.at[pl.ds(row0, rows_per_tile)],
                        acc.at[pl.ds(row0, rows_per_tile)])
        pltpu.sync_copy(ones_hbm, ones_v)
        plsc.subcore_barrier()

        def body(b, carry):
            for q in range(4):
                j = 4 * b + q
                wait_idx(q)
                pltpu.sync_copy(ones_v, acc.at[dstbs[q]], add=True)
                issue_idx(j + 4, q)
            return carry

        lax.fori_loop(0, n_chunks // 4, body, 0)
        for q in range(4):
            wait_idx(q)
        plsc.subcore_barrier()
        pltpu.sync_copy(acc.at[pl.ds(row0, rows_per_tile)],
                        out_hbm.at[c, pl.ds(row0, rows_per_tile)])

    return deg


def _make_sc_agg(n_pad, width, n_chunks, edge_split):
    mesh = plsc.VectorSubcoreMesh(core_axis_name="c", subcore_axis_name="s")
    rows_per_tile = n_pad // 16
    assert n_chunks % 8 == 0 and n_chunks >= 16

    @functools.partial(
        pl.kernel,
        out_type=jax.ShapeDtypeStruct((2, n_pad, width), jnp.float32),
        mesh=mesh,
        scratch_types=(
            [pltpu.VMEM_SHARED((n_pad, width), jnp.float32)]
            + [pltpu.VMEM((_CL,), jnp.int32) for _ in range(16)]
            + [pltpu.VMEM((4, _CL, width), jnp.float32)]
            + [pltpu.SemaphoreType.DMA for _ in range(12)]
        ),
    )
    def agg(g_hbm, src_hbm, dst_hbm, out_hbm, acc,
            sb0, sb1, sb2, sb3, sb4, sb5, sb6, sb7,
            db0, db1, db2, db3, db4, db5, db6, db7, rows_v,
            gs0, gs1, gs2, gs3,
            is0, is1, is2, is3, is4, is5, is6, is7):
        c = lax.axis_index("c")
        s = lax.axis_index("s")
        row0 = s * rows_per_tile
        gather_src = g_hbm if edge_split else g_hbm.at[c]
        init_src = (g_hbm.at[pl.ds(row0, rows_per_tile)] if edge_split
                    else g_hbm.at[c, pl.ds(row0, rows_per_tile)])
        my_src = src_hbm.at[c, s] if edge_split else src_hbm.at[s]
        my_dst = dst_hbm.at[c, s] if edge_split else dst_hbm.at[s]
        srcbs = (sb0, sb1, sb2, sb3, sb4, sb5, sb6, sb7)
        dstbs = (db0, db1, db2, db3, db4, db5, db6, db7)
        gsems = (gs0, gs1, gs2, gs3)
        isems = (is0, is1, is2, is3, is4, is5, is6, is7)

        def issue_idx(j, q):
            pltpu.async_copy(my_src.at[j], srcbs[q], isems[q])
            pltpu.async_copy(my_dst.at[j], dstbs[q], isems[q])

        def wait_idx(q):
            pltpu.make_async_copy(my_src.at[0], srcbs[q], isems[q]).wait()
            pltpu.make_async_copy(my_dst.at[0], dstbs[q], isems[q]).wait()

        def issue_gather(q, r):
            pltpu.async_copy(gather_src.at[srcbs[q]], rows_v.at[r], gsems[r])

        def wait_gather(q, r):
            pltpu.make_async_copy(gather_src.at[srcbs[q]], rows_v.at[r],
                                  gsems[r]).wait()

        for q in range(8):
            issue_idx(q, q)
        pltpu.sync_copy(init_src, acc.at[pl.ds(row0, rows_per_tile)])
        plsc.subcore_barrier()
        for j in range(4):
            wait_idx(j)
            issue_gather(j, j)

        def body(b, carry):
            for q in range(8):
                j = 8 * b + q
                r = q % 4
                wait_gather(q, r)
                pltpu.sync_copy(rows_v.at[r], acc.at[dstbs[q]], add=True)
                issue_idx(j + 8, q)
                wait_idx((q + 4) % 8)
                issue_gather((q + 4) % 8, r)
            return carry

        lax.fori_loop(0, n_chunks // 8, body, 0)
        for r in range(4):
            wait_gather(r, r)
        for q in range(4, 8):
            wait_idx(q)
        plsc.subcore_barrier()
        pltpu.sync_copy(acc.at[pl.ds(row0, rows_per_tile)],
                        out_hbm.at[c, pl.ds(row0, rows_per_tile)])

    return agg


def _tc_first(x_p, w1, cnt):
    n_pad, din = x_p.shape
    h = w1.shape[1]
    hh = h // 2
    grid = (n_pad // _BLK,)

    def body(x_ref, w_ref, cnt_ref, g_ref, dis_ref):
        cb = cnt_ref[0] + cnt_ref[1]
        disb = lax.rsqrt(cb + 1.0)
        hm = jnp.dot(x_ref[...], w_ref[...], preferred_element_type=jnp.float32)
        g_ref[0] = hm[:, :hh] * disb
        g_ref[1] = hm[:, hh:] * disb
        dis_ref[...] = disb

    return pl.pallas_call(
        body,
        grid=grid,
        in_specs=[
            pl.BlockSpec((_BLK, din), lambda i: (i, 0)),
            pl.BlockSpec((din, h), lambda i: (0, 0)),
            pl.BlockSpec((2, _BLK, _LANES), lambda i: (0, i, 0)),
        ],
        out_specs=[
            pl.BlockSpec((2, _BLK, hh), lambda i: (0, i, 0)),
            pl.BlockSpec((_BLK, _LANES), lambda i: (i, 0)),
        ],
        out_shape=[
            jax.ShapeDtypeStruct((2, n_pad, hh), jnp.float32),
            jax.ShapeDtypeStruct((n_pad, _LANES), jnp.float32),
        ],
    )(x_p, w1, cnt)


def _tc_mid(q1, dis, b1r, w2r):
    n_pad = q1.shape[1]
    hh = q1.shape[2]
    dout = w2r.shape[2]
    grid = (n_pad // _BLK,)

    def body(q_ref, dis_ref, b_ref, w_ref, g_ref):
        disb = dis_ref[...]
        z0 = jnp.maximum(q_ref[0] * disb + b_ref[0:1, :hh], 0.0)
        z1 = jnp.maximum(q_ref[1] * disb + b_ref[0:1, hh:], 0.0)
        g_ref[...] = (
            jnp.dot(z0 * disb, w_ref[0], preferred_element_type=jnp.float32)
            + jnp.dot(z1 * disb, w_ref[1], preferred_element_type=jnp.float32))

    return pl.pallas_call(
        body,
        grid=grid,
        in_specs=[
            pl.BlockSpec((2, _BLK, hh), lambda i: (0, i, 0)),
            pl.BlockSpec((_BLK, _LANES), lambda i: (i, 0)),
            pl.BlockSpec((8, 2 * hh), lambda i: (0, 0)),
            pl.BlockSpec((2, hh, dout), lambda i: (0, 0, 0)),
        ],
        out_specs=pl.BlockSpec((_BLK, dout), lambda i: (i, 0)),
        out_shape=jax.ShapeDtypeStruct((n_pad, dout), jnp.float32),
    )(q1, dis, b1r, w2r)


def _tc_last(q2, g2, dis, b2r):
    n_pad = q2.shape[1]
    dout = q2.shape[2]
    grid = (n_pad // _BLK,)

    def body(q_ref, g_ref, dis_ref, b_ref, o_ref):
        z = q_ref[0] + q_ref[1] - g_ref[...]
        z = z * dis_ref[...] + b_ref[0:1, :]
        n2 = jnp.sum(z * z, axis=1, keepdims=True)
        o_ref[...] = z / jnp.maximum(jnp.sqrt(n2), 1e-12)

    return pl.pallas_call(
        body,
        grid=grid,
        in_specs=[
            pl.BlockSpec((2, _BLK, dout), lambda i: (0, i, 0)),
            pl.BlockSpec((_BLK, dout), lambda i: (i, 0)),
            pl.BlockSpec((_BLK, _LANES), lambda i: (i, 0)),
            pl.BlockSpec((8, dout), lambda i: (0, 0)),
        ],
        out_specs=pl.BlockSpec((_BLK, dout), lambda i: (i, 0)),
        out_shape=jax.ShapeDtypeStruct((n_pad, dout), jnp.float32),
    )(q2, g2, dis, b2r)


def _chunk_edges(idx, n_parts, pad_val, lanes, round_to, n_dummy):
    e = idx.shape[0]
    chunks_pp = -(-e // (n_parts * lanes))
    chunks_pp = ((chunks_pp + round_to - 1) // round_to) * round_to
    ep = n_parts * chunks_pp * lanes
    arr = jnp.concatenate(
        [idx, jnp.full((ep - e,), pad_val, jnp.int32)]
    ).reshape(n_parts, chunks_pp, lanes)
    dummy = jnp.zeros((n_parts, n_dummy, lanes), jnp.int32)
    return jnp.concatenate([arr, dummy], axis=1), chunks_pp


def kernel(x, edge_index, W1, b1, W2, b2):
    n, din = x.shape
    e = edge_index.shape[1]
    h = W1.shape[1]
    dout = W2.shape[1]
    n_pad = ((n + _BLK - 1) // _BLK) * _BLK

    src, dst = edge_index[0], edge_index[1]

    src1, c1n = _chunk_edges(src, 16, 0, _CL, 8, 8)
    dst1, _ = _chunk_edges(dst, 16, n, _CL, 8, 8)
    src2, c2n = _chunk_edges(src, 32, 0, _CL, 8, 8)
    dst2, _ = _chunk_edges(dst, 32, n, _CL, 8, 8)
    src2 = src2.reshape(2, 16, -1, _CL)
    dst2 = dst2.reshape(2, 16, -1, _CL)
    dstd, cdn = _chunk_edges(dst, 32, n, _LANES, _KG, _KG)
    dstd = dstd.reshape(2, 16, -1, _LANES)

    x_p = jnp.pad(x, ((0, n_pad - n), (0, 0)))
    zeros_w = jnp.zeros((n_pad, _LANES), jnp.float32)
    ones_w = jnp.ones((_LANES, _LANES), jnp.float32)

    cnt = _make_sc_deg(n_pad, cdn)(dstd, zeros_w, ones_w)
    g1, dis = _tc_first(x_p, W1, cnt)
    q1 = _make_sc_agg(n_pad, h // 2, c1n, False)(g1, src1, dst1)
    g2 = _tc_mid(q1, dis,
                 jnp.broadcast_to(b1.reshape(1, h), (8, h)),
                 W2.reshape(2, h // 2, dout))
    q2 = _make_sc_agg(n_pad, dout, c2n, True)(g2, src2, dst2)
    out = _tc_last(q2, g2, dis, jnp.broadcast_to(b2.reshape(1, dout), (8, dout)))
    return out[:n]


# speedup vs baseline: 1.1745x; 1.1745x over previous
import functools

import jax
import jax.numpy as jnp
from jax import lax
from jax.experimental import pallas as pl
from jax.experimental.pallas import tpu as pltpu
from jax.experimental.pallas import tpu_sc as plsc

_LANES = 128
_CL = 120
_BLK = 256
_KG = 4


def _make_sc_deg(n_pad, n_chunks):
    mesh = plsc.VectorSubcoreMesh(core_axis_name="c", subcore_axis_name="s")
    rows_per_tile = n_pad // 16
    assert n_chunks % _KG == 0

    @functools.partial(
        pl.kernel,
        out_type=jax.ShapeDtypeStruct((2, n_pad, _LANES), jnp.float32),
        mesh=mesh,
        scratch_types=(
            [pltpu.VMEM_SHARED((n_pad, _LANES), jnp.float32)]
            + [pltpu.VMEM((_LANES,), jnp.int32) for _ in range(4)]
            + [pltpu.VMEM((_LANES, _LANES), jnp.float32)]
            + [pltpu.SemaphoreType.DMA for _ in range(4)]
        ),
    )
    def deg(dst_hbm, zeros_hbm, ones_hbm, out_hbm, acc,
            db0, db1, db2, db3, ones_v, is0, is1, is2, is3):
        c = lax.axis_index("c")
        s = lax.axis_index("s")
        row0 = s * rows_per_tile
        my_dst = dst_hbm.at[c, s]
        dstbs = (db0, db1, db2, db3)
        isems = (is0, is1, is2, is3)

        def issue_idx(j, q):
            pltpu.async_copy(my_dst.at[j], dstbs[q], isems[q])

        def wait_idx(q):
            pltpu.make_async_copy(my_dst.at[0], dstbs[q], isems[q]).wait()

        for q in range(4):
            issue_idx(q, q)
        pltpu.sync_copy(zeros_hbm.at[pl.ds(row0, rows_per_tile)],
                        acc.at[pl.ds(row0, rows_per_tile)])
        pltpu.sync_copy(ones_hbm, ones_v)
        plsc.subcore_barrier()

        def body(b, carry):
            for q in range(4):
                j = 4 * b + q
                wait_idx(q)
                pltpu.sync_copy(ones_v, acc.at[dstbs[q]], add=True)
                issue_idx(j + 4, q)
            return carry

        lax.fori_loop(0, n_chunks // 4, body, 0)
        for q in range(4):
            wait_idx(q)
        plsc.subcore_barrier()
        pltpu.sync_copy(acc.at[pl.ds(row0, rows_per_tile)],
                        out_hbm.at[c, pl.ds(row0, rows_per_tile)])

    return deg


def _make_sc_agg(n_acc, width, n_chunks, edge_split):
    mesh = plsc.VectorSubcoreMesh(core_axis_name="c", subcore_axis_name="s")
    rows_per_tile = n_acc // 16
    assert n_chunks % 6 == 0 and n_chunks >= 12

    @functools.partial(
        pl.kernel,
        out_type=jax.ShapeDtypeStruct((2, n_acc, width), jnp.float32),
        mesh=mesh,
        scratch_types=(
            [pltpu.VMEM_SHARED((n_acc, width), jnp.float32)]
            + [pltpu.VMEM((_CL,), jnp.int32) for _ in range(12)]
            + [pltpu.VMEM((3, _CL, width), jnp.float32)]
            + [pltpu.SemaphoreType.DMA for _ in range(9)]
        ),
    )
    def agg(g_hbm, src_hbm, dst_hbm, out_hbm, acc,
            sb0, sb1, sb2, sb3, sb4, sb5,
            db0, db1, db2, db3, db4, db5, rows_v,
            gs0, gs1, gs2, is0, is1, is2, is3, is4, is5):
        c = lax.axis_index("c")
        s = lax.axis_index("s")
        row0 = s * rows_per_tile
        gather_src = g_hbm if edge_split else g_hbm.at[c]
        init_src = (g_hbm.at[pl.ds(row0, rows_per_tile)] if edge_split
                    else g_hbm.at[c, pl.ds(row0, rows_per_tile)])
        my_src = src_hbm.at[c, s] if edge_split else src_hbm.at[s]
        my_dst = dst_hbm.at[c, s] if edge_split else dst_hbm.at[s]
        srcbs = (sb0, sb1, sb2, sb3, sb4, sb5)
        dstbs = (db0, db1, db2, db3, db4, db5)
        gsems = (gs0, gs1, gs2)
        isems = (is0, is1, is2, is3, is4, is5)

        def issue_idx(j, q):
            pltpu.async_copy(my_src.at[j], srcbs[q], isems[q])
            pltpu.async_copy(my_dst.at[j], dstbs[q], isems[q])

        def wait_idx(q):
            pltpu.make_async_copy(my_src.at[0], srcbs[q], isems[q]).wait()
            pltpu.make_async_copy(my_dst.at[0], dstbs[q], isems[q]).wait()

        def issue_gather(q, r):
            pltpu.async_copy(gather_src.at[srcbs[q]], rows_v.at[r], gsems[r])

        def wait_gather(q, r):
            pltpu.make_async_copy(gather_src.at[srcbs[q]], rows_v.at[r],
                                  gsems[r]).wait()

        for q in range(6):
            issue_idx(q, q)
        pltpu.sync_copy(init_src, acc.at[pl.ds(row0, rows_per_tile)])
        plsc.subcore_barrier()
        for j in range(3):
            wait_idx(j)
            issue_gather(j, j)

        def body(b, carry):
            for q in range(6):
                j = 6 * b + q
                r = q % 3
                wait_gather(q, r)
                pltpu.sync_copy(rows_v.at[r], acc.at[dstbs[q]], add=True)
                issue_idx(j + 6, q)
                wait_idx((q + 3) % 6)
                issue_gather((q + 3) % 6, r)
            return carry

        lax.fori_loop(0, n_chunks // 6, body, 0)
        for r in range(3):
            wait_gather(r, r)
        for q in range(3, 6):
            wait_idx(q)
        plsc.subcore_barrier()
        pltpu.sync_copy(acc.at[pl.ds(row0, rows_per_tile)],
                        out_hbm.at[c, pl.ds(row0, rows_per_tile)])

    return agg


def _tc_first(x_p, w1, cnt):
    n_pad, din = x_p.shape
    h = w1.shape[1]
    hh = h // 2
    grid = (n_pad // _BLK,)

    def body(x_ref, w_ref, cnt_ref, g_ref, dis_ref):
        cb = cnt_ref[0] + cnt_ref[1]
        disb = lax.rsqrt(cb + 1.0)
        hm = jnp.dot(x_ref[...], w_ref[...], preferred_element_type=jnp.float32)
        g_ref[0] = hm[:, :hh] * disb
        g_ref[1] = hm[:, hh:] * disb
        dis_ref[...] = disb

    return pl.pallas_call(
        body,
        grid=grid,
        in_specs=[
            pl.BlockSpec((_BLK, din), lambda i: (i, 0)),
            pl.BlockSpec((din, h), lambda i: (0, 0)),
            pl.BlockSpec((2, _BLK, _LANES), lambda i: (0, i, 0)),
        ],
        out_specs=[
            pl.BlockSpec((2, _BLK, hh), lambda i: (0, i, 0)),
            pl.BlockSpec((_BLK, _LANES), lambda i: (i, 0)),
        ],
        out_shape=[
            jax.ShapeDtypeStruct((2, n_pad, hh), jnp.float32),
            jax.ShapeDtypeStruct((n_pad, _LANES), jnp.float32),
        ],
    )(x_p, w1, cnt)


def _tc_mid(q1, dis, b1r, w2r):
    n_pad = q1.shape[1]
    hh = q1.shape[2]
    dout = w2r.shape[2]
    grid = (n_pad // _BLK,)

    def body(q_ref, dis_ref, b_ref, w_ref, g_ref):
        disb = dis_ref[...]
        z0 = jnp.maximum(q_ref[0] * disb + b_ref[0:1, :hh], 0.0)
        z1 = jnp.maximum(q_ref[1] * disb + b_ref[0:1, hh:], 0.0)
        g_ref[...] = (
            jnp.dot(z0 * disb, w_ref[0], preferred_element_type=jnp.float32)
            + jnp.dot(z1 * disb, w_ref[1], preferred_element_type=jnp.float32))

    return pl.pallas_call(
        body,
        grid=grid,
        in_specs=[
            pl.BlockSpec((2, _BLK, hh), lambda i: (0, i, 0)),
            pl.BlockSpec((_BLK, _LANES), lambda i: (i, 0)),
            pl.BlockSpec((8, 2 * hh), lambda i: (0, 0)),
            pl.BlockSpec((2, hh, dout), lambda i: (0, 0, 0)),
        ],
        out_specs=pl.BlockSpec((_BLK, dout), lambda i: (i, 0)),
        out_shape=jax.ShapeDtypeStruct((n_pad, dout), jnp.float32),
    )(q1, dis, b1r, w2r)


def _tc_last(q2, g2, dis, b2r):
    n_pad = q2.shape[1]
    dout = q2.shape[2]
    grid = (n_pad // _BLK,)

    def body(q_ref, g_ref, dis_ref, b_ref, o_ref):
        z = q_ref[0] + q_ref[1] - g_ref[...]
        z = z * dis_ref[...] + b_ref[0:1, :]
        n2 = jnp.sum(z * z, axis=1, keepdims=True)
        o_ref[...] = z / jnp.maximum(jnp.sqrt(n2), 1e-12)

    return pl.pallas_call(
        body,
        grid=grid,
        in_specs=[
            pl.BlockSpec((2, _BLK, dout), lambda i: (0, i, 0)),
            pl.BlockSpec((_BLK, dout), lambda i: (i, 0)),
            pl.BlockSpec((_BLK, _LANES), lambda i: (i, 0)),
            pl.BlockSpec((8, dout), lambda i: (0, 0)),
        ],
        out_specs=pl.BlockSpec((_BLK, dout), lambda i: (i, 0)),
        out_shape=jax.ShapeDtypeStruct((n_pad, dout), jnp.float32),
    )(q2, g2, dis, b2r)


def _chunk_edges(idx, n_parts, pad_val, lanes, round_to, n_dummy):
    e = idx.shape[0]
    chunks_pp = -(-e // (n_parts * lanes))
    chunks_pp = ((chunks_pp + round_to - 1) // round_to) * round_to
    ep = n_parts * chunks_pp * lanes
    arr = jnp.concatenate(
        [idx, jnp.full((ep - e,), pad_val, jnp.int32)]
    ).reshape(n_parts, chunks_pp, lanes)
    dummy = jnp.zeros((n_parts, n_dummy, lanes), jnp.int32)
    return jnp.concatenate([arr, dummy], axis=1), chunks_pp


def kernel(x, edge_index, W1, b1, W2, b2):
    n, din = x.shape
    e = edge_index.shape[1]
    h = W1.shape[1]
    dout = W2.shape[1]
    n_pad = ((n + _BLK - 1) // _BLK) * _BLK

    src, dst = edge_index[0], edge_index[1]

    n_acc = 10112
    src1, c1n = _chunk_edges(src, 16, 0, _CL, 6, 6)
    dst1, _ = _chunk_edges(dst, 16, n, _CL, 6, 6)
    src2, c2n = _chunk_edges(src, 32, 0, _CL, 6, 6)
    dst2, _ = _chunk_edges(dst, 32, n, _CL, 6, 6)
    src2 = src2.reshape(2, 16, -1, _CL)
    dst2 = dst2.reshape(2, 16, -1, _CL)
    dstd, cdn = _chunk_edges(dst, 32, n, _LANES, _KG, _KG)
    dstd = dstd.reshape(2, 16, -1, _LANES)

    x_p = jnp.pad(x, ((0, n_pad - n), (0, 0)))
    zeros_w = jnp.zeros((n_pad, _LANES), jnp.float32)
    ones_w = jnp.ones((_LANES, _LANES), jnp.float32)

    cnt = _make_sc_deg(n_pad, cdn)(dstd, zeros_w, ones_w)
    g1, dis = _tc_first(x_p, W1, cnt)
    q1 = _make_sc_agg(n_acc, h // 2, c1n, False)(g1, src1, dst1)
    q1 = jnp.pad(q1, ((0, 0), (0, n_pad - n_acc), (0, 0)))
    g2 = _tc_mid(q1, dis,
                 jnp.broadcast_to(b1.reshape(1, h), (8, h)),
                 W2.reshape(2, h // 2, dout))
    q2 = _make_sc_agg(n_acc, dout, c2n, True)(g2, src2, dst2)
    q2 = jnp.pad(q2, ((0, 0), (0, n_pad - n_acc), (0, 0)))
    out = _tc_last(q2, g2, dis, jnp.broadcast_to(b2.reshape(1, dout), (8, dout)))
    return out[:n]


# speedup vs baseline: 1.2275x; 1.0451x over previous
import functools

import jax
import jax.numpy as jnp
from jax import lax
from jax.experimental import pallas as pl
from jax.experimental.pallas import tpu as pltpu
from jax.experimental.pallas import tpu_sc as plsc

_LANES = 128
_BLK = 256
_KG = 4


def _make_sc_deg(n_pad, n_chunks):
    mesh = plsc.VectorSubcoreMesh(core_axis_name="c", subcore_axis_name="s")
    rows_per_tile = n_pad // 16
    assert n_chunks % _KG == 0

    @functools.partial(
        pl.kernel,
        out_type=jax.ShapeDtypeStruct((2, n_pad, _LANES), jnp.float32),
        mesh=mesh,
        scratch_types=(
            [pltpu.VMEM_SHARED((n_pad, _LANES), jnp.float32)]
            + [pltpu.VMEM((_LANES,), jnp.int32) for _ in range(4)]
            + [pltpu.VMEM((_LANES, _LANES), jnp.float32)]
            + [pltpu.SemaphoreType.DMA for _ in range(4)]
        ),
    )
    def deg(dst_hbm, zeros_hbm, ones_hbm, out_hbm, acc,
            db0, db1, db2, db3, ones_v, is0, is1, is2, is3):
        c = lax.axis_index("c")
        s = lax.axis_index("s")
        row0 = s * rows_per_tile
        my_dst = dst_hbm.at[c, s]
        dstbs = (db0, db1, db2, db3)
        isems = (is0, is1, is2, is3)

        def issue_idx(j, q):
            pltpu.async_copy(my_dst.at[j], dstbs[q], isems[q])

        def wait_idx(q):
            pltpu.make_async_copy(my_dst.at[0], dstbs[q], isems[q]).wait()

        for q in range(4):
            issue_idx(q, q)
        pltpu.sync_copy(zeros_hbm.at[pl.ds(row0, rows_per_tile)],
                        acc.at[pl.ds(row0, rows_per_tile)])
        pltpu.sync_copy(ones_hbm, ones_v)
        plsc.subcore_barrier()

        def body(b, carry):
            for q in range(4):
                j = 4 * b + q
                wait_idx(q)
                pltpu.sync_copy(ones_v, acc.at[dstbs[q]], add=True)
                issue_idx(j + 4, q)
            return carry

        lax.fori_loop(0, n_chunks // 4, body, 0)
        for q in range(4):
            wait_idx(q)
        plsc.subcore_barrier()
        pltpu.sync_copy(acc.at[pl.ds(row0, rows_per_tile)],
                        out_hbm.at[c, pl.ds(row0, rows_per_tile)])

    return deg


def _make_sc_agg(n_pad, width, n_chunks, edge_split):
    mesh = plsc.VectorSubcoreMesh(core_axis_name="c", subcore_axis_name="s")
    rows_per_tile = n_pad // 16
    assert n_chunks % _KG == 0 and n_chunks >= 8

    @functools.partial(
        pl.kernel,
        out_type=jax.ShapeDtypeStruct((2, n_pad, width), jnp.float32),
        mesh=mesh,
        scratch_types=(
            [pltpu.VMEM_SHARED((n_pad, width), jnp.float32)]
            + [pltpu.VMEM((_LANES,), jnp.int32) for _ in range(8)]
            + [pltpu.VMEM((2, _LANES, width), jnp.float32)]
            + [pltpu.SemaphoreType.DMA for _ in range(6)]
        ),
    )
    def agg(g_hbm, src_hbm, dst_hbm, out_hbm, acc,
            sb0, sb1, sb2, sb3, db0, db1, db2, db3, rows_v,
            gs0, gs1, is0, is1, is2, is3):
        c = lax.axis_index("c")
        s = lax.axis_index("s")
        row0 = s * rows_per_tile
        gather_src = g_hbm if edge_split else g_hbm.at[c]
        init_src = (g_hbm.at[pl.ds(row0, rows_per_tile)] if edge_split
                    else g_hbm.at[c, pl.ds(row0, rows_per_tile)])
        my_src = src_hbm.at[c, s] if edge_split else src_hbm.at[s]
        my_dst = dst_hbm.at[c, s] if edge_split else dst_hbm.at[s]
        srcbs = (sb0, sb1, sb2, sb3)
        dstbs = (db0, db1, db2, db3)
        gsems = (gs0, gs1)
        isems = (is0, is1, is2, is3)

        def issue_idx(j, q):
            pltpu.async_copy(my_src.at[j], srcbs[q], isems[q])
            pltpu.async_copy(my_dst.at[j], dstbs[q], isems[q])

        def wait_idx(q):
            pltpu.make_async_copy(my_src.at[0], srcbs[q], isems[q]).wait()
            pltpu.make_async_copy(my_dst.at[0], dstbs[q], isems[q]).wait()

        def issue_gather(q, p):
            pltpu.async_copy(gather_src.at[srcbs[q]], rows_v.at[p], gsems[p])

        def wait_gather(q, p):
            pltpu.make_async_copy(gather_src.at[srcbs[q]], rows_v.at[p],
                                  gsems[p]).wait()

        for q in range(4):
            issue_idx(q, q)
        pltpu.sync_copy(init_src, acc.at[pl.ds(row0, rows_per_tile)])
        plsc.subcore_barrier()
        for j in range(2):
            wait_idx(j)
            issue_gather(j, j)

        def body(b, carry):
            for q in range(4):
                j = 4 * b + q
                p = q % 2
                wait_gather(q, p)
                pltpu.sync_copy(rows_v.at[p], acc.at[dstbs[q]], add=True)
                issue_idx(j + 4, q)
                wait_idx((q + 2) % 4)
                issue_gather((q + 2) % 4, p)
            return carry

        lax.fori_loop(0, n_chunks // 4 - 1, body, 0)
        for q in range(4):
            p = q % 2
            wait_gather(q, p)
            pltpu.sync_copy(rows_v.at[p], acc.at[dstbs[q]], add=True)
            if q < 2:
                wait_idx(q + 2)
                issue_gather(q + 2, p)
        plsc.subcore_barrier()
        pltpu.sync_copy(acc.at[pl.ds(row0, rows_per_tile)],
                        out_hbm.at[c, pl.ds(row0, rows_per_tile)])

    return agg


def _tc_first(x_p, w1, cnt):
    n_pad, din = x_p.shape
    h = w1.shape[1]
    hh = h // 2
    grid = (n_pad // _BLK,)

    def body(x_ref, w_ref, cnt_ref, g_ref, dis_ref):
        cb = cnt_ref[0] + cnt_ref[1]
        disb = lax.rsqrt(cb + 1.0)
        hm = jnp.dot(x_ref[...], w_ref[...], preferred_element_type=jnp.float32)
        g_ref[0] = hm[:, :hh] * disb
        g_ref[1] = hm[:, hh:] * disb
        dis_ref[...] = disb

    return pl.pallas_call(
        body,
        grid=grid,
        in_specs=[
            pl.BlockSpec((_BLK, din), lambda i: (i, 0)),
            pl.BlockSpec((din, h), lambda i: (0, 0)),
            pl.BlockSpec((2, _BLK, _LANES), lambda i: (0, i, 0)),
        ],
        out_specs=[
            pl.BlockSpec((2, _BLK, hh), lambda i: (0, i, 0)),
            pl.BlockSpec((_BLK, _LANES), lambda i: (i, 0)),
        ],
        out_shape=[
            jax.ShapeDtypeStruct((2, n_pad, hh), jnp.float32),
            jax.ShapeDtypeStruct((n_pad, _LANES), jnp.float32),
        ],
    )(x_p, w1, cnt)


def _tc_mid(q1, dis, b1r, w2r):
    n_pad = q1.shape[1]
    hh = q1.shape[2]
    dout = w2r.shape[2]
    grid = (n_pad // _BLK,)

    def body(q_ref, dis_ref, b_ref, w_ref, g_ref):
        disb = dis_ref[...]
        z0 = jnp.maximum(q_ref[0] * disb + b_ref[0:1, :hh], 0.0)
        z1 = jnp.maximum(q_ref[1] * disb + b_ref[0:1, hh:], 0.0)
        g_ref[...] = (
            jnp.dot(z0 * disb, w_ref[0], preferred_element_type=jnp.float32)
            + jnp.dot(z1 * disb, w_ref[1], preferred_element_type=jnp.float32))

    return pl.pallas_call(
        body,
        grid=grid,
        in_specs=[
            pl.BlockSpec((2, _BLK, hh), lambda i: (0, i, 0)),
            pl.BlockSpec((_BLK, _LANES), lambda i: (i, 0)),
            pl.BlockSpec((8, 2 * hh), lambda i: (0, 0)),
            pl.BlockSpec((2, hh, dout), lambda i: (0, 0, 0)),
        ],
        out_specs=pl.BlockSpec((_BLK, dout), lambda i: (i, 0)),
        out_shape=jax.ShapeDtypeStruct((n_pad, dout), jnp.float32),
    )(q1, dis, b1r, w2r)


def _tc_last(q2, g2, dis, b2r):
    n_pad = q2.shape[1]
    dout = q2.shape[2]
    grid = (n_pad // _BLK,)

    def body(q_ref, g_ref, dis_ref, b_ref, o_ref):
        z = q_ref[0] + q_ref[1] - g_ref[...]
        z = z * dis_ref[...] + b_ref[0:1, :]
        n2 = jnp.sum(z * z, axis=1, keepdims=True)
        o_ref[...] = z / jnp.maximum(jnp.sqrt(n2), 1e-12)

    return pl.pallas_call(
        body,
        grid=grid,
        in_specs=[
            pl.BlockSpec((2, _BLK, dout), lambda i: (0, i, 0)),
            pl.BlockSpec((_BLK, dout), lambda i: (i, 0)),
            pl.BlockSpec((_BLK, _LANES), lambda i: (i, 0)),
            pl.BlockSpec((8, dout), lambda i: (0, 0)),
        ],
        out_specs=pl.BlockSpec((_BLK, dout), lambda i: (i, 0)),
        out_shape=jax.ShapeDtypeStruct((n_pad, dout), jnp.float32),
    )(q2, g2, dis, b2r)


def _chunk_edges(idx, n_parts, pad_val):
    e = idx.shape[0]
    chunks_pp = -(-e // (n_parts * _LANES))
    chunks_pp = ((chunks_pp + _KG - 1) // _KG) * _KG
    ep = n_parts * chunks_pp * _LANES
    arr = jnp.concatenate(
        [idx, jnp.full((ep - e,), pad_val, jnp.int32)]
    ).reshape(n_parts, chunks_pp, _LANES)
    dummy = jnp.zeros((n_parts, _KG, _LANES), jnp.int32)
    return jnp.concatenate([arr, dummy], axis=1), chunks_pp


def kernel(x, edge_index, W1, b1, W2, b2):
    n, din = x.shape
    e = edge_index.shape[1]
    h = W1.shape[1]
    dout = W2.shape[1]
    n_pad = ((n + _BLK - 1) // _BLK) * _BLK

    src, dst = edge_index[0], edge_index[1]

    src1, c1n = _chunk_edges(src, 16, 0)
    dst1, _ = _chunk_edges(dst, 16, n)
    src2, c2n = _chunk_edges(src, 32, 0)
    dst2, _ = _chunk_edges(dst, 32, n)
    src2 = src2.reshape(2, 16, -1, _LANES)
    dst2 = dst2.reshape(2, 16, -1, _LANES)

    x_p = jnp.pad(x, ((0, n_pad - n), (0, 0)))
    zeros_w = jnp.zeros((n_pad, _LANES), jnp.float32)
    ones_w = jnp.ones((_LANES, _LANES), jnp.float32)

    cnt = _make_sc_deg(n_pad, c2n)(dst2, zeros_w, ones_w)
    g1, dis = _tc_first(x_p, W1, cnt)
    q1 = _make_sc_agg(n_pad, h // 2, c1n, False)(g1, src1, dst1)
    g2 = _tc_mid(q1, dis,
                 jnp.broadcast_to(b1.reshape(1, h), (8, h)),
                 W2.reshape(2, h // 2, dout))
    q2 = _make_sc_agg(n_pad, dout, c2n, True)(g2, src2, dst2)
    out = _tc_last(q2, g2, dis, jnp.broadcast_to(b2.reshape(1, dout), (8, dout)))
    return out[:n]
